# Initial kernel scaffold; baseline (speedup 1.0000x reference)
import jax
import jax.numpy as jnp
from jax.experimental import pallas as pl


def kernel(edge_attr, W0, W1, W2):
    raise NotImplementedError("write your pallas kernel here")


# speedup vs baseline: 1.0889x; 1.0889x over previous
import functools

import jax
import jax.numpy as jnp
from jax import lax
from jax.experimental import pallas as pl
from jax.experimental.pallas import tpu as pltpu
from jax.experimental.pallas import tpu_sc as plsc

F0, F1, F2 = 5, 6, 2
EMB = 128
E = 320000
NROWS = F0 * F1 * F2

NC, NS = 2, 16
NW = NC * NS
CHUNK = 128
TOTAL_CHUNKS = E // CHUNK


def _table_body(w0_ref, w1_ref, w2_ref, c_ref):
    r = lax.broadcasted_iota(jnp.int32, (NROWS, 1), 0)
    i0 = r // (F1 * F2)
    i1 = (r // F2) % F1
    i2 = r % F2
    oh0 = (i0 == lax.broadcasted_iota(jnp.int32, (NROWS, F0), 1)).astype(jnp.float32)
    oh1 = (i1 == lax.broadcasted_iota(jnp.int32, (NROWS, F1), 1)).astype(jnp.float32)
    oh2 = (i2 == lax.broadcasted_iota(jnp.int32, (NROWS, F2), 1)).astype(jnp.float32)
    acc = jnp.dot(oh0, w0_ref[...], preferred_element_type=jnp.float32)
    acc = acc + jnp.dot(oh1, w1_ref[...], preferred_element_type=jnp.float32)
    acc = acc + jnp.dot(oh2, w2_ref[...], preferred_element_type=jnp.float32)
    c_ref[...] = acc


def _build_table(w0, w1, w2):
    return pl.pallas_call(
        _table_body,
        out_shape=jax.ShapeDtypeStruct((NROWS, EMB), jnp.float32),
    )(w0, w1, w2)


def _sc_body(a0_hbm, a1_hbm, a2_hbm, c_hbm, out_hbm, a0_v, a1_v, a2_v, idx_v, rows_v, sem):
    wid = lax.axis_index("s") * NC + lax.axis_index("c")
    nchunks = (TOTAL_CHUNKS - wid + NW - 1) // NW

    def step(k, carry):
        c = wid + k * NW
        base = c * CHUNK
        pltpu.sync_copy(a0_hbm.at[pl.ds(base, CHUNK)], a0_v)
        pltpu.sync_copy(a1_hbm.at[pl.ds(base, CHUNK)], a1_v)
        pltpu.sync_copy(a2_hbm.at[pl.ds(base, CHUNK)], a2_v)
        for i in range(CHUNK // 16):
            s = pl.ds(i * 16, 16)
            idx_v[s] = a0_v[s] * (F1 * F2) + a1_v[s] * F2 + a2_v[s]
        pltpu.async_copy(c_hbm.at[idx_v], rows_v, sem).wait()
        pltpu.sync_copy(rows_v, out_hbm.at[pl.ds(base, CHUNK)])
        return carry

    lax.fori_loop(0, nchunks, step, 0)


@functools.partial(jax.jit, static_argnames=())
def _sc_lookup(a0, a1, a2, table):
    mesh = plsc.VectorSubcoreMesh(core_axis_name="c", subcore_axis_name="s")
    fn = pl.kernel(
        _sc_body,
        out_type=jax.ShapeDtypeStruct((E, EMB), jnp.float32),
        mesh=mesh,
        scratch_types=[
            pltpu.VMEM((CHUNK,), jnp.int32),
            pltpu.VMEM((CHUNK,), jnp.int32),
            pltpu.VMEM((CHUNK,), jnp.int32),
            pltpu.VMEM((CHUNK,), jnp.int32),
            pltpu.VMEM((CHUNK, EMB), jnp.float32),
            pltpu.SemaphoreType.DMA,
        ],
    )
    return fn(a0, a1, a2, table)


def kernel(edge_attr, W0, W1, W2):
    table = _build_table(W0, W1, W2)
    ea = jnp.asarray(edge_attr, jnp.int32)
    return _sc_lookup(ea[:, 0], ea[:, 1], ea[:, 2], table)
